# baseline (speedup 1.0000x reference)
import functools

import jax
import jax.numpy as jnp
from jax import lax
from jax.experimental import pallas as pl
from jax.experimental.pallas import tpu as pltpu
from jax.experimental.pallas import tpu_sc as plsc

N = 10000
E = 320000
NC = 2
NS = 16
NW = NC * NS
CHUNK = 128
IDXB = 8
EPAD = -(-E // (NW * CHUNK * 2 * IDXB)) * (NW * CHUNK * 2 * IDXB)
NCH = EPAD // (NW * CHUNK)
NBLK = NCH // IDXB
NPAIR = NBLK // 2
NP = N + 112
STRIPE = NP // NS
DEGW = 16
M_BLK = 2000


def _mesh():
    return plsc.VectorSubcoreMesh(core_axis_name="c", subcore_axis_name="s")


ZROWS = 64


def _make_scatter(D, P0, P1):
    assert 16 * NS * (P0 + P1) == EPAD // CHUNK
    NPART = NC if P1 > 0 else 1

    @functools.partial(
        pl.kernel,
        out_type=jax.ShapeDtypeStruct((NPART, NP, D), jnp.float32),
        mesh=_mesh(),
        compiler_params=pltpu.CompilerParams(use_tc_tiling_on_sc=False),
        scratch_types=[
            pltpu.VMEM((2, IDXB, CHUNK), jnp.int32),
            pltpu.VMEM((2, IDXB, CHUNK), jnp.int32),
            pltpu.VMEM((2, CHUNK, D), jnp.float32),
            pltpu.VMEM((ZROWS, D), jnp.float32),
            pltpu.VMEM_SHARED((NP, D), jnp.float32),
            pltpu.SemaphoreType.DMA,
            pltpu.SemaphoreType.DMA,
            pltpu.SemaphoreType.DMA,
        ],
    )
    def scatter_kernel(src_hbm, dst_hbm, g_hbm, zero_hbm, out_hbm,
                       sidx, didx, rows, zbuf, acc, isem, gsem, ssem):
        c = lax.axis_index("c")
        s = lax.axis_index("s")

        def gather_start(idx_slice, b):
            pltpu.async_copy(g_hbm.at[idx_slice], rows.at[b], gsem)

        def gather_wait(b):
            pltpu.make_async_copy(g_hbm.at[sidx.at[0, 0]], rows.at[b],
                                  gsem).wait()

        def scatter_start(b, idx_slice):
            pltpu.async_copy(rows.at[b], acc.at[idx_slice], ssem, add=True)

        def scatter_wait():
            pltpu.make_async_copy(rows.at[0], acc.at[didx.at[0, 0]],
                                  ssem).wait()

        def body(P, base):
            pltpu.sync_copy(zero_hbm.at[pl.ds(0, ZROWS)], zbuf)
            nfull = STRIPE // ZROWS
            for z in range(nfull):
                pltpu.sync_copy(zbuf, acc.at[pl.ds(s * STRIPE + z * ZROWS,
                                                   ZROWS)])
            rem = STRIPE - nfull * ZROWS
            if rem:
                pltpu.sync_copy(zbuf.at[pl.ds(0, rem)],
                                acc.at[pl.ds(s * STRIPE + nfull * ZROWS, rem)])
            pltpu.sync_copy(src_hbm.at[pl.ds(base, IDXB)], sidx.at[0])
            pltpu.sync_copy(dst_hbm.at[pl.ds(base, IDXB)], didx.at[0])
            pltpu.async_copy(src_hbm.at[pl.ds(base + IDXB, IDXB)],
                             sidx.at[1], isem)
            pltpu.async_copy(dst_hbm.at[pl.ds(base + IDXB, IDXB)],
                             didx.at[1], isem)
            gather_start(sidx.at[0, 0], 0)
            plsc.subcore_barrier()

            def wait_idx(bb):
                pltpu.make_async_copy(src_hbm.at[pl.ds(0, IDXB)],
                                      sidx.at[bb], isem).wait()
                pltpu.make_async_copy(dst_hbm.at[pl.ds(0, IDXB)],
                                      didx.at[bb], isem).wait()

            def prefetch_idx(blk, bb):
                off = pl.multiple_of(base + blk * IDXB, IDXB)
                pltpu.async_copy(src_hbm.at[pl.ds(off, IDXB)],
                                 sidx.at[bb], isem)
                pltpu.async_copy(dst_hbm.at[pl.ds(off, IDXB)],
                                 didx.at[bb], isem)

            def half(blk, nblk, bb):
                for off in range(IDXB):
                    b = off % 2
                    gather_wait(b)
                    if off > 0:
                        scatter_wait()
                    if off < IDXB - 1:
                        gather_start(sidx.at[bb, off + 1], 1 - b)
                        scatter_start(b, didx.at[bb, off])
                    else:
                        scatter_start(b, didx.at[bb, off])

                        @pl.when(blk + 1 < nblk)
                        def _():
                            wait_idx(1 - bb)
                            gather_start(sidx.at[1 - bb, 0], 1 - b)
                            scatter_wait()

                            @pl.when(blk + 2 < nblk)
                            def _():
                                prefetch_idx(blk + 2, bb)

            def pairfn(i, carry):
                half(2 * i, 2 * P, 0)
                half(2 * i + 1, 2 * P, 1)
                return carry

            lax.fori_loop(0, P, pairfn, 0)
            scatter_wait()
            plsc.subcore_barrier()
            pltpu.sync_copy(acc.at[pl.ds(s * STRIPE, STRIPE)],
                            out_hbm.at[jnp.minimum(c, NPART - 1),
                                       pl.ds(s * STRIPE, STRIPE)])

        if P1 > 0:
            P = jnp.where(c == 0, P0, P1)
            base = jnp.where(c == 0, s * (16 * P0), 256 * P0 + s * (16 * P1))
            body(P, pl.multiple_of(base, IDXB))
        else:
            @pl.when(c == 0)
            def _():
                body(jnp.int32(P0), pl.multiple_of(s * (16 * P0), IDXB))

    return scatter_kernel


_scatter128 = _make_scatter(128, 10, 0)
_scatter64 = _make_scatter(64, 10, 0)
_scatter16 = _make_scatter(DEGW, 6, 4)


def _dinv(deg_ref):
    deg = deg_ref[0, :, 0:1] + 1.0
    for k in range(1, deg_ref.shape[0]):
        deg = deg + deg_ref[k, :, 0:1]
    return lax.rsqrt(deg)


def _tc1_body(x_ref, w_ref, deg_ref, g_ref):
    d = _dinv(deg_ref)
    g_ref[...] = d * jnp.dot(x_ref[...], w_ref[...],
                             preferred_element_type=jnp.float32)


def _tc1(x, W1, degp):
    return pl.pallas_call(
        _tc1_body,
        grid=(N // M_BLK,),
        in_specs=[
            pl.BlockSpec((M_BLK, 128), lambda i: (i, 0)),
            pl.BlockSpec((128, 128), lambda i: (0, 0)),
            pl.BlockSpec((NC, M_BLK, DEGW), lambda i: (0, i, 0)),
        ],
        out_specs=pl.BlockSpec((M_BLK, 128), lambda i: (i, 0)),
        out_shape=jax.ShapeDtypeStruct((N, 128), jnp.float32),
    )(x, W1, degp)


def _psum(p_ref):
    acc = p_ref[0]
    for k in range(1, p_ref.shape[0]):
        acc = acc + p_ref[k]
    return acc


def _tc2(p1, g1, degp, b1, W2):
    PC = p1.shape[0]

    def body(p_ref, g_ref, deg_ref, b_ref, w_ref, o_ref):
        d = _dinv(deg_ref)
        h = d * (_psum(p_ref) + g_ref[...]) + b_ref[...]
        h = jnp.maximum(h, 0.0)
        o_ref[...] = d * jnp.dot(h, w_ref[...],
                                 preferred_element_type=jnp.float32)

    return pl.pallas_call(
        body,
        grid=(N // M_BLK,),
        in_specs=[
            pl.BlockSpec((PC, M_BLK, 128), lambda i: (0, i, 0)),
            pl.BlockSpec((M_BLK, 128), lambda i: (i, 0)),
            pl.BlockSpec((NC, M_BLK, DEGW), lambda i: (0, i, 0)),
            pl.BlockSpec((1, 128), lambda i: (0, 0)),
            pl.BlockSpec((128, 64), lambda i: (0, 0)),
        ],
        out_specs=pl.BlockSpec((M_BLK, 64), lambda i: (i, 0)),
        out_shape=jax.ShapeDtypeStruct((N, 64), jnp.float32),
    )(p1, g1, degp, b1.reshape(1, 128), W2)


def _tc3(p2, g2, degp, b2):
    PC = p2.shape[0]

    def body(p_ref, g_ref, deg_ref, b_ref, o_ref):
        d = _dinv(deg_ref)
        h = d * (_psum(p_ref) + g_ref[...]) + b_ref[...]
        o_ref[...] = jnp.maximum(h, 0.0)

    return pl.pallas_call(
        body,
        grid=(N // M_BLK,),
        in_specs=[
            pl.BlockSpec((PC, M_BLK, 64), lambda i: (0, i, 0)),
            pl.BlockSpec((M_BLK, 64), lambda i: (i, 0)),
            pl.BlockSpec((NC, M_BLK, DEGW), lambda i: (0, i, 0)),
            pl.BlockSpec((1, 64), lambda i: (0, 0)),
        ],
        out_specs=pl.BlockSpec((M_BLK, 64), lambda i: (i, 0)),
        out_shape=jax.ShapeDtypeStruct((N, 64), jnp.float32),
    )(p2, g2, degp, b2.reshape(1, 64))


def kernel(x, edge_index, W1, b1, W2, b2):
    x = x.astype(jnp.float32)
    ei = edge_index.astype(jnp.int32)
    pad = EPAD - E
    src3 = jnp.concatenate([ei[0], jnp.zeros((pad,), jnp.int32)]
                           ).reshape(EPAD // CHUNK, CHUNK)
    dst3 = jnp.concatenate([ei[1], jnp.full((pad,), N, jnp.int32)]
                           ).reshape(EPAD // CHUNK, CHUNK)
    ones16 = jnp.ones((NP, DEGW), jnp.float32)
    zdeg = jnp.zeros((NP, DEGW), jnp.float32)
    z128 = jnp.zeros((NP, 128), jnp.float32)
    z64 = jnp.zeros((NP, 64), jnp.float32)

    degp = _scatter16(src3, dst3, ones16, zdeg)
    g1 = _tc1(x, W1, degp)
    p1 = _scatter128(src3, dst3, g1, z128)
    g2 = _tc2(p1, g1, degp, b1, W2)
    p2 = _scatter64(src3, dst3, g2, z64)
    return _tc3(p2, g2, degp, b2)


# speedup vs baseline: 1.4099x; 1.4099x over previous
import functools

import jax
import jax.numpy as jnp
from jax import lax
from jax.experimental import pallas as pl
from jax.experimental.pallas import tpu as pltpu
from jax.experimental.pallas import tpu_sc as plsc

N = 10000
E = 320000
NC = 2
NS = 16
NW = NC * NS
CHUNK = 128
IDXB = 8
EPAD = -(-E // (NW * CHUNK * 2 * IDXB)) * (NW * CHUNK * 2 * IDXB)
NCH = EPAD // (NW * CHUNK)
NBLK = NCH // IDXB
NPAIR = NBLK // 2
NP = N + 112
STRIPE = NP // NS
DEGW = 16
M_BLK = 2000


def _mesh():
    return plsc.VectorSubcoreMesh(core_axis_name="c", subcore_axis_name="s")


ZROWS = 64


def _make_scatter(D, P0, P1):
    assert 16 * NS * (P0 + P1) == EPAD // CHUNK
    NPART = NC if P1 > 0 else 1

    @functools.partial(
        pl.kernel,
        out_type=jax.ShapeDtypeStruct((NPART, NP, D), jnp.float32),
        mesh=_mesh(),
        compiler_params=pltpu.CompilerParams(use_tc_tiling_on_sc=False),
        scratch_types=[
            pltpu.VMEM((2, IDXB, CHUNK), jnp.int32),
            pltpu.VMEM((2, IDXB, CHUNK), jnp.int32),
            pltpu.VMEM((2, CHUNK, D), jnp.float32),
            pltpu.VMEM((ZROWS, D), jnp.float32),
            pltpu.VMEM_SHARED((NP, D), jnp.float32),
            pltpu.SemaphoreType.DMA,
            pltpu.SemaphoreType.DMA,
            pltpu.SemaphoreType.DMA,
        ],
    )
    def scatter_kernel(src_hbm, dst_hbm, g_hbm, zero_hbm, out_hbm,
                       sidx, didx, rows, zbuf, acc, isem, gsem, ssem):
        c = lax.axis_index("c")
        s = lax.axis_index("s")

        def gather_start(idx_slice, b):
            pltpu.async_copy(g_hbm.at[idx_slice], rows.at[b], gsem)

        def gather_wait(b):
            pltpu.make_async_copy(g_hbm.at[sidx.at[0, 0]], rows.at[b],
                                  gsem).wait()

        def scatter_start(b, idx_slice):
            pltpu.async_copy(rows.at[b], acc.at[idx_slice], ssem, add=True)

        def scatter_wait():
            pltpu.make_async_copy(rows.at[0], acc.at[didx.at[0, 0]],
                                  ssem).wait()

        def body(P, base):
            pltpu.sync_copy(zero_hbm.at[pl.ds(0, ZROWS)], zbuf)
            nfull = STRIPE // ZROWS
            for z in range(nfull):
                pltpu.sync_copy(zbuf, acc.at[pl.ds(s * STRIPE + z * ZROWS,
                                                   ZROWS)])
            rem = STRIPE - nfull * ZROWS
            if rem:
                pltpu.sync_copy(zbuf.at[pl.ds(0, rem)],
                                acc.at[pl.ds(s * STRIPE + nfull * ZROWS, rem)])
            pltpu.sync_copy(src_hbm.at[pl.ds(base, IDXB)], sidx.at[0])
            pltpu.sync_copy(dst_hbm.at[pl.ds(base, IDXB)], didx.at[0])
            pltpu.async_copy(src_hbm.at[pl.ds(base + IDXB, IDXB)],
                             sidx.at[1], isem)
            pltpu.async_copy(dst_hbm.at[pl.ds(base + IDXB, IDXB)],
                             didx.at[1], isem)
            gather_start(sidx.at[0, 0], 0)
            plsc.subcore_barrier()

            def wait_idx(bb):
                pltpu.make_async_copy(src_hbm.at[pl.ds(0, IDXB)],
                                      sidx.at[bb], isem).wait()
                pltpu.make_async_copy(dst_hbm.at[pl.ds(0, IDXB)],
                                      didx.at[bb], isem).wait()

            def prefetch_idx(blk, bb):
                off = pl.multiple_of(base + blk * IDXB, IDXB)
                pltpu.async_copy(src_hbm.at[pl.ds(off, IDXB)],
                                 sidx.at[bb], isem)
                pltpu.async_copy(dst_hbm.at[pl.ds(off, IDXB)],
                                 didx.at[bb], isem)

            def half(blk, nblk, bb):
                for off in range(IDXB):
                    b = off % 2
                    gather_wait(b)
                    if off > 0:
                        scatter_wait()
                    if off < IDXB - 1:
                        gather_start(sidx.at[bb, off + 1], 1 - b)
                        scatter_start(b, didx.at[bb, off])
                    else:
                        scatter_start(b, didx.at[bb, off])

                        @pl.when(blk + 1 < nblk)
                        def _():
                            wait_idx(1 - bb)
                            gather_start(sidx.at[1 - bb, 0], 1 - b)
                            scatter_wait()

                            @pl.when(blk + 2 < nblk)
                            def _():
                                prefetch_idx(blk + 2, bb)

            def pairfn(i, carry):
                half(2 * i, 2 * P, 0)
                half(2 * i + 1, 2 * P, 1)
                return carry

            lax.fori_loop(0, P, pairfn, 0)
            scatter_wait()
            plsc.subcore_barrier()
            pltpu.sync_copy(acc.at[pl.ds(s * STRIPE, STRIPE)],
                            out_hbm.at[jnp.minimum(c, NPART - 1),
                                       pl.ds(s * STRIPE, STRIPE)])

        if P1 > 0:
            P = jnp.where(c == 0, P0, P1)
            base = jnp.where(c == 0, s * (16 * P0), 256 * P0 + s * (16 * P1))
            body(P, pl.multiple_of(base, IDXB))
        else:
            @pl.when(c == 0)
            def _():
                body(jnp.int32(P0), pl.multiple_of(s * (16 * P0), IDXB))

    return scatter_kernel


def _make_scatter_bf16(D, P0, P1):
    assert 16 * NS * (P0 + P1) == EPAD // CHUNK
    HC = CHUNK // 2
    NG = D // 32

    @functools.partial(
        pl.kernel,
        out_type=jax.ShapeDtypeStruct((NC, NP, D), jnp.float32),
        mesh=_mesh(),
        compiler_params=pltpu.CompilerParams(use_tc_tiling_on_sc=False,
                                             needs_layout_passes=False),
        scratch_types=[
            pltpu.VMEM((2, IDXB, CHUNK), jnp.int32),
            pltpu.VMEM((2, 2 * IDXB, HC), jnp.int32),
            pltpu.VMEM((2, CHUNK, D), jnp.bfloat16),
            pltpu.VMEM((2, HC, D), jnp.float32),
            pltpu.VMEM((ZROWS, D), jnp.float32),
            pltpu.VMEM_SHARED((NP, D), jnp.float32),
            pltpu.SemaphoreType.DMA,
            pltpu.SemaphoreType.DMA,
            pltpu.SemaphoreType.DMA,
        ],
    )
    def scatter_kernel(src_hbm, dst_hbm, g_hbm, zero_hbm, out_hbm,
                       sidx, didx, bfrows, frows, zbuf, acc, isem, gsem, ssem):
        c = lax.axis_index("c")
        s = lax.axis_index("s")

        def gather_start(idx_slice, b):
            pltpu.async_copy(g_hbm.at[idx_slice], bfrows.at[b], gsem)

        def gather_wait(b):
            pltpu.make_async_copy(g_hbm.at[sidx.at[0, 0]], bfrows.at[b],
                                  gsem).wait()

        def scatter_start(q, idx_slice):
            pltpu.async_copy(frows.at[q], acc.at[idx_slice], ssem, add=True)

        def scatter_wait():
            pltpu.make_async_copy(frows.at[0], acc.at[didx.at[0, 0]],
                                  ssem).wait()

        def convert_half(b, q):
            def conv(r4, carry):
                for k in range(4):
                    row = r4 * 4 + k
                    for g in range(NG):
                        v = bfrows[b, q * HC + row, pl.ds(32 * g, 32)]
                        x, y = plsc.unpack(
                            v, format=plsc.PackFormat.INTERLEAVED)
                        frows[q, row, pl.ds(32 * g, 16)] = x
                        frows[q, row, pl.ds(32 * g + 16, 16)] = y
                return carry

            lax.fori_loop(0, HC // 4, conv, 0)

        def body(P, base):
            pltpu.sync_copy(zero_hbm.at[pl.ds(0, ZROWS)], zbuf)
            nfull = STRIPE // ZROWS
            for z in range(nfull):
                pltpu.sync_copy(zbuf, acc.at[pl.ds(s * STRIPE + z * ZROWS,
                                                   ZROWS)])
            rem = STRIPE - nfull * ZROWS
            if rem:
                pltpu.sync_copy(zbuf.at[pl.ds(0, rem)],
                                acc.at[pl.ds(s * STRIPE + nfull * ZROWS, rem)])
            pltpu.sync_copy(src_hbm.at[pl.ds(base, IDXB)], sidx.at[0])
            pltpu.sync_copy(dst_hbm.at[pl.ds(2 * base, 2 * IDXB)], didx.at[0])
            pltpu.async_copy(src_hbm.at[pl.ds(base + IDXB, IDXB)],
                             sidx.at[1], isem)
            pltpu.async_copy(dst_hbm.at[pl.ds(2 * base + 2 * IDXB, 2 * IDXB)],
                             didx.at[1], isem)
            gather_start(sidx.at[0, 0], 0)
            plsc.subcore_barrier()

            def wait_idx(bb):
                pltpu.make_async_copy(src_hbm.at[pl.ds(0, IDXB)],
                                      sidx.at[bb], isem).wait()
                pltpu.make_async_copy(dst_hbm.at[pl.ds(0, 2 * IDXB)],
                                      didx.at[bb], isem).wait()

            def prefetch_idx(blk, bb):
                off = pl.multiple_of(base + blk * IDXB, IDXB)
                pltpu.async_copy(src_hbm.at[pl.ds(off, IDXB)],
                                 sidx.at[bb], isem)
                pltpu.async_copy(dst_hbm.at[pl.ds(2 * off, 2 * IDXB)],
                                 didx.at[bb], isem)

            def half(blk, nblk, bb):
                for off in range(IDXB):
                    b = off % 2
                    gather_wait(b)
                    if off > 0:
                        scatter_wait()
                        scatter_wait()
                    if off < IDXB - 1:
                        gather_start(sidx.at[bb, off + 1], 1 - b)
                        convert_half(b, 0)
                        scatter_start(0, didx.at[bb, 2 * off])
                        convert_half(b, 1)
                        scatter_start(1, didx.at[bb, 2 * off + 1])
                    else:
                        @pl.when(blk + 1 < nblk)
                        def _():
                            wait_idx(1 - bb)
                            gather_start(sidx.at[1 - bb, 0], 1 - b)

                        convert_half(b, 0)
                        scatter_start(0, didx.at[bb, 2 * off])
                        convert_half(b, 1)
                        scatter_start(1, didx.at[bb, 2 * off + 1])

                        @pl.when(blk + 1 < nblk)
                        def _():
                            scatter_wait()
                            scatter_wait()

                            @pl.when(blk + 2 < nblk)
                            def _():
                                prefetch_idx(blk + 2, bb)

            def pairfn(i, carry):
                half(2 * i, 2 * P, 0)
                half(2 * i + 1, 2 * P, 1)
                return carry

            lax.fori_loop(0, P, pairfn, 0)
            scatter_wait()
            scatter_wait()
            plsc.subcore_barrier()
            pltpu.sync_copy(acc.at[pl.ds(s * STRIPE, STRIPE)],
                            out_hbm.at[c, pl.ds(s * STRIPE, STRIPE)])

        P = jnp.where(c == 0, P0, P1)
        base = jnp.where(c == 0, s * (16 * P0), 256 * P0 + s * (16 * P1))
        body(P, pl.multiple_of(base, IDXB))

    return scatter_kernel


def _bfperm(g):
    n, d = g.shape
    gb = g.astype(jnp.bfloat16).reshape(n, d // 32, 2, 16)
    return gb.transpose(0, 1, 3, 2).reshape(n, d)


_scatter128 = _make_scatter_bf16(128, 8, 2)
_scatter64 = _make_scatter_bf16(64, 7, 3)
_scatter16 = _make_scatter(DEGW, 6, 4)


def _dinv(deg_ref):
    deg = deg_ref[0, :, 0:1] + 1.0
    for k in range(1, deg_ref.shape[0]):
        deg = deg + deg_ref[k, :, 0:1]
    return lax.rsqrt(deg)


def _tc1_body(x_ref, w_ref, deg_ref, g_ref):
    d = _dinv(deg_ref)
    g_ref[...] = d * jnp.dot(x_ref[...], w_ref[...],
                             preferred_element_type=jnp.float32)


def _tc1(x, W1, degp):
    return pl.pallas_call(
        _tc1_body,
        grid=(N // M_BLK,),
        in_specs=[
            pl.BlockSpec((M_BLK, 128), lambda i: (i, 0)),
            pl.BlockSpec((128, 128), lambda i: (0, 0)),
            pl.BlockSpec((NC, M_BLK, DEGW), lambda i: (0, i, 0)),
        ],
        out_specs=pl.BlockSpec((M_BLK, 128), lambda i: (i, 0)),
        out_shape=jax.ShapeDtypeStruct((N, 128), jnp.float32),
    )(x, W1, degp)


def _psum(p_ref):
    acc = p_ref[0]
    for k in range(1, p_ref.shape[0]):
        acc = acc + p_ref[k]
    return acc


def _tc2(p1, g1, degp, b1, W2):
    PC = p1.shape[0]

    def body(p_ref, g_ref, deg_ref, b_ref, w_ref, o_ref):
        d = _dinv(deg_ref)
        h = d * (_psum(p_ref) + g_ref[...]) + b_ref[...]
        h = jnp.maximum(h, 0.0)
        o_ref[...] = d * jnp.dot(h, w_ref[...],
                                 preferred_element_type=jnp.float32)

    return pl.pallas_call(
        body,
        grid=(N // M_BLK,),
        in_specs=[
            pl.BlockSpec((PC, M_BLK, 128), lambda i: (0, i, 0)),
            pl.BlockSpec((M_BLK, 128), lambda i: (i, 0)),
            pl.BlockSpec((NC, M_BLK, DEGW), lambda i: (0, i, 0)),
            pl.BlockSpec((1, 128), lambda i: (0, 0)),
            pl.BlockSpec((128, 64), lambda i: (0, 0)),
        ],
        out_specs=pl.BlockSpec((M_BLK, 64), lambda i: (i, 0)),
        out_shape=jax.ShapeDtypeStruct((N, 64), jnp.float32),
    )(p1, g1, degp, b1.reshape(1, 128), W2)


def _tc3(p2, g2, degp, b2):
    PC = p2.shape[0]

    def body(p_ref, g_ref, deg_ref, b_ref, o_ref):
        d = _dinv(deg_ref)
        h = d * (_psum(p_ref) + g_ref[...]) + b_ref[...]
        o_ref[...] = jnp.maximum(h, 0.0)

    return pl.pallas_call(
        body,
        grid=(N // M_BLK,),
        in_specs=[
            pl.BlockSpec((PC, M_BLK, 64), lambda i: (0, i, 0)),
            pl.BlockSpec((M_BLK, 64), lambda i: (i, 0)),
            pl.BlockSpec((NC, M_BLK, DEGW), lambda i: (0, i, 0)),
            pl.BlockSpec((1, 64), lambda i: (0, 0)),
        ],
        out_specs=pl.BlockSpec((M_BLK, 64), lambda i: (i, 0)),
        out_shape=jax.ShapeDtypeStruct((N, 64), jnp.float32),
    )(p2, g2, degp, b2.reshape(1, 64))


def kernel(x, edge_index, W1, b1, W2, b2):
    x = x.astype(jnp.float32)
    ei = edge_index.astype(jnp.int32)
    pad = EPAD - E
    src3 = jnp.concatenate([ei[0], jnp.zeros((pad,), jnp.int32)]
                           ).reshape(EPAD // CHUNK, CHUNK)
    dst_flat = jnp.concatenate([ei[1], jnp.full((pad,), N, jnp.int32)])
    dst3 = dst_flat.reshape(EPAD // CHUNK, CHUNK)
    dst2 = dst_flat.reshape(EPAD // CHUNK * 2, CHUNK // 2)
    ones16 = jnp.ones((NP, DEGW), jnp.float32)
    zdeg = jnp.zeros((NP, DEGW), jnp.float32)
    z128 = jnp.zeros((NP, 128), jnp.float32)
    z64 = jnp.zeros((NP, 64), jnp.float32)

    degp = _scatter16(src3, dst3, ones16, zdeg)
    g1 = _tc1(x, W1, degp)
    p1 = _scatter128(src3, dst2, _bfperm(g1), z128)
    g2 = _tc2(p1, g1, degp, b1, W2)
    p2 = _scatter64(src3, dst2, _bfperm(g2), z64)
    return _tc3(p2, g2, degp, b2)


# speedup vs baseline: 1.6301x; 1.1562x over previous
import functools

import jax
import jax.numpy as jnp
from jax import lax
from jax.experimental import pallas as pl
from jax.experimental.pallas import tpu as pltpu
from jax.experimental.pallas import tpu_sc as plsc

N = 10000
E = 320000
NC = 2
NS = 16
NW = NC * NS
CHUNK = 128
IDXB = 8
EPAD = -(-E // (NW * CHUNK * 2 * IDXB)) * (NW * CHUNK * 2 * IDXB)
NCH = EPAD // (NW * CHUNK)
NBLK = NCH // IDXB
NPAIR = NBLK // 2
NP = N + 112
STRIPE = NP // NS
DEGW = 16
M_BLK = 2000


def _mesh():
    return plsc.VectorSubcoreMesh(core_axis_name="c", subcore_axis_name="s")


ZROWS = 64


def _make_scatter(D, P0, P1):
    assert 16 * NS * (P0 + P1) == EPAD // CHUNK
    NPART = NC if P1 > 0 else 1

    @functools.partial(
        pl.kernel,
        out_type=jax.ShapeDtypeStruct((NPART, NP, D), jnp.float32),
        mesh=_mesh(),
        compiler_params=pltpu.CompilerParams(use_tc_tiling_on_sc=False),
        scratch_types=[
            pltpu.VMEM((2, IDXB, CHUNK), jnp.int32),
            pltpu.VMEM((2, IDXB, CHUNK), jnp.int32),
            pltpu.VMEM((2, CHUNK, D), jnp.float32),
            pltpu.VMEM((ZROWS, D), jnp.float32),
            pltpu.VMEM_SHARED((NP, D), jnp.float32),
            pltpu.SemaphoreType.DMA,
            pltpu.SemaphoreType.DMA,
            pltpu.SemaphoreType.DMA,
        ],
    )
    def scatter_kernel(src_hbm, dst_hbm, g_hbm, zero_hbm, out_hbm,
                       sidx, didx, rows, zbuf, acc, isem, gsem, ssem):
        c = lax.axis_index("c")
        s = lax.axis_index("s")

        def gather_start(idx_slice, b):
            pltpu.async_copy(g_hbm.at[idx_slice], rows.at[b], gsem)

        def gather_wait(b):
            pltpu.make_async_copy(g_hbm.at[sidx.at[0, 0]], rows.at[b],
                                  gsem).wait()

        def scatter_start(b, idx_slice):
            pltpu.async_copy(rows.at[b], acc.at[idx_slice], ssem, add=True)

        def scatter_wait():
            pltpu.make_async_copy(rows.at[0], acc.at[didx.at[0, 0]],
                                  ssem).wait()

        def body(P, base):
            pltpu.sync_copy(zero_hbm.at[pl.ds(0, ZROWS)], zbuf)
            nfull = STRIPE // ZROWS
            for z in range(nfull):
                pltpu.sync_copy(zbuf, acc.at[pl.ds(s * STRIPE + z * ZROWS,
                                                   ZROWS)])
            rem = STRIPE - nfull * ZROWS
            if rem:
                pltpu.sync_copy(zbuf.at[pl.ds(0, rem)],
                                acc.at[pl.ds(s * STRIPE + nfull * ZROWS, rem)])
            pltpu.sync_copy(src_hbm.at[pl.ds(base, IDXB)], sidx.at[0])
            pltpu.sync_copy(dst_hbm.at[pl.ds(base, IDXB)], didx.at[0])
            pltpu.async_copy(src_hbm.at[pl.ds(base + IDXB, IDXB)],
                             sidx.at[1], isem)
            pltpu.async_copy(dst_hbm.at[pl.ds(base + IDXB, IDXB)],
                             didx.at[1], isem)
            gather_start(sidx.at[0, 0], 0)
            plsc.subcore_barrier()

            def wait_idx(bb):
                pltpu.make_async_copy(src_hbm.at[pl.ds(0, IDXB)],
                                      sidx.at[bb], isem).wait()
                pltpu.make_async_copy(dst_hbm.at[pl.ds(0, IDXB)],
                                      didx.at[bb], isem).wait()

            def prefetch_idx(blk, bb):
                off = pl.multiple_of(base + blk * IDXB, IDXB)
                pltpu.async_copy(src_hbm.at[pl.ds(off, IDXB)],
                                 sidx.at[bb], isem)
                pltpu.async_copy(dst_hbm.at[pl.ds(off, IDXB)],
                                 didx.at[bb], isem)

            def half(blk, nblk, bb):
                for off in range(IDXB):
                    b = off % 2
                    gather_wait(b)
                    if off > 0:
                        scatter_wait()
                    if off < IDXB - 1:
                        gather_start(sidx.at[bb, off + 1], 1 - b)
                        scatter_start(b, didx.at[bb, off])
                    else:
                        scatter_start(b, didx.at[bb, off])

                        @pl.when(blk + 1 < nblk)
                        def _():
                            wait_idx(1 - bb)
                            gather_start(sidx.at[1 - bb, 0], 1 - b)
                            scatter_wait()

                            @pl.when(blk + 2 < nblk)
                            def _():
                                prefetch_idx(blk + 2, bb)

            def pairfn(i, carry):
                half(2 * i, 2 * P, 0)
                half(2 * i + 1, 2 * P, 1)
                return carry

            lax.fori_loop(0, P, pairfn, 0)
            scatter_wait()
            plsc.subcore_barrier()
            pltpu.sync_copy(acc.at[pl.ds(s * STRIPE, STRIPE)],
                            out_hbm.at[jnp.minimum(c, NPART - 1),
                                       pl.ds(s * STRIPE, STRIPE)])

        if P1 > 0:
            P = jnp.where(c == 0, P0, P1)
            base = jnp.where(c == 0, s * (16 * P0), 256 * P0 + s * (16 * P1))
            body(P, pl.multiple_of(base, IDXB))
        else:
            @pl.when(c == 0)
            def _():
                body(jnp.int32(P0), pl.multiple_of(s * (16 * P0), IDXB))

    return scatter_kernel


def _make_scatter_bf16(D, P0, P1):
    assert 16 * NS * (P0 + P1) == EPAD // CHUNK
    HC = CHUNK // 2
    NG = D // 32

    @functools.partial(
        pl.kernel,
        out_type=jax.ShapeDtypeStruct((NC, NP, D), jnp.float32),
        mesh=_mesh(),
        compiler_params=pltpu.CompilerParams(use_tc_tiling_on_sc=False,
                                             needs_layout_passes=False),
        scratch_types=[
            pltpu.VMEM((2, IDXB, CHUNK), jnp.int32),
            pltpu.VMEM((2, 2 * IDXB, HC), jnp.int32),
            pltpu.VMEM((2, CHUNK, D), jnp.bfloat16),
            pltpu.VMEM((2, HC, D), jnp.float32),
            pltpu.VMEM((ZROWS, D), jnp.float32),
            pltpu.VMEM_SHARED((NP, D), jnp.float32),
            pltpu.SemaphoreType.DMA,
            pltpu.SemaphoreType.DMA,
            pltpu.SemaphoreType.DMA,
        ],
    )
    def scatter_kernel(src_hbm, dst_hbm, g_hbm, zero_hbm, out_hbm,
                       sidx, didx, bfrows, frows, zbuf, acc, isem, gsem, ssem):
        c = lax.axis_index("c")
        s = lax.axis_index("s")

        def gather_start(idx_slice, b):
            pltpu.async_copy(g_hbm.at[idx_slice], bfrows.at[b], gsem)

        def gather_wait(b):
            pltpu.make_async_copy(g_hbm.at[sidx.at[0, 0]], bfrows.at[b],
                                  gsem).wait()

        def scatter_start(q, idx_slice):
            pltpu.async_copy(frows.at[q], acc.at[idx_slice], ssem, add=True)

        def scatter_wait():
            pltpu.make_async_copy(frows.at[0], acc.at[didx.at[0, 0]],
                                  ssem).wait()

        def convert_half(b, q):
            def conv(r4, carry):
                for k in range(4):
                    row = r4 * 4 + k
                    for g in range(NG):
                        v = bfrows[b, q * HC + row, pl.ds(32 * g, 32)]
                        x, y = plsc.unpack(
                            v, format=plsc.PackFormat.INTERLEAVED)
                        frows[q, row, pl.ds(32 * g, 16)] = x
                        frows[q, row, pl.ds(32 * g + 16, 16)] = y
                return carry

            lax.fori_loop(0, HC // 4, conv, 0)

        def body(P, base):
            pltpu.sync_copy(zero_hbm.at[pl.ds(0, ZROWS)], zbuf)
            nfull = STRIPE // ZROWS
            for z in range(nfull):
                pltpu.sync_copy(zbuf, acc.at[pl.ds(s * STRIPE + z * ZROWS,
                                                   ZROWS)])
            rem = STRIPE - nfull * ZROWS
            if rem:
                pltpu.sync_copy(zbuf.at[pl.ds(0, rem)],
                                acc.at[pl.ds(s * STRIPE + nfull * ZROWS, rem)])
            pltpu.sync_copy(src_hbm.at[pl.ds(base, IDXB)], sidx.at[0])
            pltpu.sync_copy(dst_hbm.at[pl.ds(2 * base, 2 * IDXB)], didx.at[0])
            pltpu.async_copy(src_hbm.at[pl.ds(base + IDXB, IDXB)],
                             sidx.at[1], isem)
            pltpu.async_copy(dst_hbm.at[pl.ds(2 * base + 2 * IDXB, 2 * IDXB)],
                             didx.at[1], isem)
            gather_start(sidx.at[0, 0], 0)
            plsc.subcore_barrier()

            def wait_idx(bb):
                pltpu.make_async_copy(src_hbm.at[pl.ds(0, IDXB)],
                                      sidx.at[bb], isem).wait()
                pltpu.make_async_copy(dst_hbm.at[pl.ds(0, 2 * IDXB)],
                                      didx.at[bb], isem).wait()

            def prefetch_idx(blk, bb):
                off = pl.multiple_of(base + blk * IDXB, IDXB)
                pltpu.async_copy(src_hbm.at[pl.ds(off, IDXB)],
                                 sidx.at[bb], isem)
                pltpu.async_copy(dst_hbm.at[pl.ds(2 * off, 2 * IDXB)],
                                 didx.at[bb], isem)

            def half(blk, nblk, bb):
                for off in range(IDXB):
                    b = off % 2
                    gather_wait(b)
                    if off > 0:
                        scatter_wait()
                        scatter_wait()
                    if off < IDXB - 1:
                        gather_start(sidx.at[bb, off + 1], 1 - b)
                        convert_half(b, 0)
                        scatter_start(0, didx.at[bb, 2 * off])
                        convert_half(b, 1)
                        scatter_start(1, didx.at[bb, 2 * off + 1])
                    else:
                        @pl.when(blk + 1 < nblk)
                        def _():
                            wait_idx(1 - bb)
                            gather_start(sidx.at[1 - bb, 0], 1 - b)

                        convert_half(b, 0)
                        scatter_start(0, didx.at[bb, 2 * off])
                        convert_half(b, 1)
                        scatter_start(1, didx.at[bb, 2 * off + 1])

                        @pl.when(blk + 1 < nblk)
                        def _():
                            scatter_wait()
                            scatter_wait()

                            @pl.when(blk + 2 < nblk)
                            def _():
                                prefetch_idx(blk + 2, bb)

            def pairfn(i, carry):
                half(2 * i, 2 * P, 0)
                half(2 * i + 1, 2 * P, 1)
                return carry

            lax.fori_loop(0, P, pairfn, 0)
            scatter_wait()
            scatter_wait()
            plsc.subcore_barrier()
            pltpu.sync_copy(acc.at[pl.ds(s * STRIPE, STRIPE)],
                            out_hbm.at[c, pl.ds(s * STRIPE, STRIPE)])

        P = jnp.where(c == 0, P0, P1)
        base = jnp.where(c == 0, s * (16 * P0), 256 * P0 + s * (16 * P1))
        body(P, pl.multiple_of(base, IDXB))

    return scatter_kernel


def _bfperm(g):
    n, d = g.shape
    gb = g.astype(jnp.bfloat16).reshape(n, d // 32, 2, 16)
    return gb.transpose(0, 1, 3, 2).reshape(n, d)


_scatter128 = _make_scatter_bf16(128, 6, 4)
_scatter64 = _make_scatter_bf16(64, 6, 4)
_scatter16 = _make_scatter(DEGW, 6, 4)


def _dinv(deg_ref):
    deg = deg_ref[0, :, 0:1] + 1.0
    for k in range(1, deg_ref.shape[0]):
        deg = deg + deg_ref[k, :, 0:1]
    return lax.rsqrt(deg)


def _tc1_body(x_ref, w_ref, deg_ref, g_ref):
    d = _dinv(deg_ref)
    g_ref[...] = d * jnp.dot(x_ref[...], w_ref[...],
                             preferred_element_type=jnp.float32)


def _tc1(x, W1, degp):
    return pl.pallas_call(
        _tc1_body,
        grid=(N // M_BLK,),
        in_specs=[
            pl.BlockSpec((M_BLK, 128), lambda i: (i, 0)),
            pl.BlockSpec((128, 128), lambda i: (0, 0)),
            pl.BlockSpec((NC, M_BLK, DEGW), lambda i: (0, i, 0)),
        ],
        out_specs=pl.BlockSpec((M_BLK, 128), lambda i: (i, 0)),
        out_shape=jax.ShapeDtypeStruct((N, 128), jnp.float32),
    )(x, W1, degp)


def _psum(p_ref):
    acc = p_ref[0]
    for k in range(1, p_ref.shape[0]):
        acc = acc + p_ref[k]
    return acc


def _tc2(p1, g1, degp, b1, W2):
    PC = p1.shape[0]

    def body(p_ref, g_ref, deg_ref, b_ref, w_ref, o_ref):
        d = _dinv(deg_ref)
        h = d * (_psum(p_ref) + g_ref[...]) + b_ref[...]
        h = jnp.maximum(h, 0.0)
        o_ref[...] = d * jnp.dot(h, w_ref[...],
                                 preferred_element_type=jnp.float32)

    return pl.pallas_call(
        body,
        grid=(N // M_BLK,),
        in_specs=[
            pl.BlockSpec((PC, M_BLK, 128), lambda i: (0, i, 0)),
            pl.BlockSpec((M_BLK, 128), lambda i: (i, 0)),
            pl.BlockSpec((NC, M_BLK, DEGW), lambda i: (0, i, 0)),
            pl.BlockSpec((1, 128), lambda i: (0, 0)),
            pl.BlockSpec((128, 64), lambda i: (0, 0)),
        ],
        out_specs=pl.BlockSpec((M_BLK, 64), lambda i: (i, 0)),
        out_shape=jax.ShapeDtypeStruct((N, 64), jnp.float32),
    )(p1, g1, degp, b1.reshape(1, 128), W2)


def _tc3(p2, g2, degp, b2):
    PC = p2.shape[0]

    def body(p_ref, g_ref, deg_ref, b_ref, o_ref):
        d = _dinv(deg_ref)
        h = d * (_psum(p_ref) + g_ref[...]) + b_ref[...]
        o_ref[...] = jnp.maximum(h, 0.0)

    return pl.pallas_call(
        body,
        grid=(N // M_BLK,),
        in_specs=[
            pl.BlockSpec((PC, M_BLK, 64), lambda i: (0, i, 0)),
            pl.BlockSpec((M_BLK, 64), lambda i: (i, 0)),
            pl.BlockSpec((NC, M_BLK, DEGW), lambda i: (0, i, 0)),
            pl.BlockSpec((1, 64), lambda i: (0, 0)),
        ],
        out_specs=pl.BlockSpec((M_BLK, 64), lambda i: (i, 0)),
        out_shape=jax.ShapeDtypeStruct((N, 64), jnp.float32),
    )(p2, g2, degp, b2.reshape(1, 64))


def kernel(x, edge_index, W1, b1, W2, b2):
    x = x.astype(jnp.float32)
    ei = edge_index.astype(jnp.int32)
    pad = EPAD - E
    src3 = jnp.concatenate([ei[0], jnp.zeros((pad,), jnp.int32)]
                           ).reshape(EPAD // CHUNK, CHUNK)
    dst_flat = jnp.concatenate([ei[1], jnp.full((pad,), N, jnp.int32)])
    dst3 = dst_flat.reshape(EPAD // CHUNK, CHUNK)
    dst2 = dst_flat.reshape(EPAD // CHUNK * 2, CHUNK // 2)
    ones16 = jnp.ones((NP, DEGW), jnp.float32)
    zdeg = jnp.zeros((NP, DEGW), jnp.float32)
    z128 = jnp.zeros((NP, 128), jnp.float32)
    z64 = jnp.zeros((NP, 64), jnp.float32)

    degp = _scatter16(src3, dst3, ones16, zdeg)
    g1 = _tc1(x, W1, degp)
    p1 = _scatter128(src3, dst2, _bfperm(g1), z128)
    g2 = _tc2(p1, g1, degp, b1, W2)
    p2 = _scatter64(src3, dst2, _bfperm(g2), z64)
    return _tc3(p2, g2, degp, b2)
